# Initial kernel scaffold; baseline (speedup 1.0000x reference)
import jax
import jax.numpy as jnp
from jax.experimental import pallas as pl


def kernel(drug_graph, protein_graph, drug_embedding, protein_embedding, gene_embedding, W_gate, b_gate, W_exp, b_exp, W_out0, b_out0, W_out1, b_out1, W_out2, b_out2, W_int, b_int):
    raise NotImplementedError("write your pallas kernel here")


# speedup vs baseline: 1.1151x; 1.1151x over previous
import functools

import jax
import jax.numpy as jnp
from jax.experimental import pallas as pl
from jax.experimental.pallas import tpu as pltpu

B = 1024
D = 8960
E = 6
H = 512
DC = 1280
NC = D // DC


def _moe_kernel(x_ref, wg_ref, bg_ref, wexp_ref, bexp_ref,
                w0_ref, b0_ref, w1_ref, b1_ref, w2_ref, b2_ref,
                wi_ref, bi_ref,
                probs_ref, pred_ref,
                xb_ref, logits_ref, acc_ref, fused_ref, wsel_ref):
    e = pl.program_id(0)
    c = pl.program_id(1)

    @pl.when(e == 0)
    def _gate_pass():
        x_blk = x_ref[...]
        xb_blk = x_blk.astype(jnp.bfloat16)
        xb_ref[:, pl.ds(c * DC, DC)] = xb_blk
        part = jnp.dot(xb_blk, wg_ref[...].astype(jnp.bfloat16),
                       preferred_element_type=jnp.float32)
        prev = jnp.where(c == 0, jnp.zeros_like(logits_ref), logits_ref[...])
        logits_ref[...] = prev + part

    xb_c = xb_ref[:, pl.ds(c * DC, DC)]
    w_blk = wexp_ref[0].astype(jnp.bfloat16)
    prod = jnp.dot(xb_c, w_blk, preferred_element_type=jnp.float32)
    prev = jnp.where(c == 0, jnp.zeros_like(acc_ref), acc_ref[...])
    acc_ref[...] = prev + prod

    @pl.when(jnp.logical_and(e == 0, c == NC - 1))
    def _gate_epilogue():
        logits = logits_ref[...] + bg_ref[...]
        m = jnp.max(logits, axis=-1, keepdims=True)
        ex = jnp.exp(logits - m)
        probs = ex / jnp.sum(ex, axis=-1, keepdims=True)
        probs_ref[...] = probs
        iota = jax.lax.broadcasted_iota(jnp.int32, (B, E), 1)
        v1 = jnp.max(probs, axis=-1, keepdims=True)
        i1 = jnp.min(jnp.where(probs == v1, iota, E), axis=-1, keepdims=True)
        masked = jnp.where(iota == i1, -jnp.inf, probs)
        v2 = jnp.max(masked, axis=-1, keepdims=True)
        i2 = jnp.min(jnp.where(masked == v2, iota, E), axis=-1, keepdims=True)
        denom = v1 + v2 + 1e-9
        wsel_ref[...] = (jnp.where(iota == i1, v1 / denom, 0.0)
                         + jnp.where(iota == i2, v2 / denom, 0.0))

    @pl.when(c == NC - 1)
    def _expert_epilogue():
        row_iota = jax.lax.broadcasted_iota(jnp.int32, (E, H), 0)
        b_row = jnp.sum(jnp.where(row_iota == e, bexp_ref[...], 0.0),
                        axis=0, keepdims=True)
        eo = jnp.maximum(acc_ref[...] + b_row, 0.0)
        col_iota = jax.lax.broadcasted_iota(jnp.int32, (B, E), 1)
        w_col = jnp.sum(jnp.where(col_iota == e, wsel_ref[...], 0.0),
                        axis=-1, keepdims=True)
        prev = jnp.where(e == 0, jnp.zeros_like(fused_ref), fused_ref[...])
        fused_ref[...] = prev + eo * w_col

    @pl.when(jnp.logical_and(e == E - 1, c == NC - 1))
    def _mlp():
        bf = jnp.bfloat16
        f = fused_ref[...].astype(bf)
        h = jnp.tanh(jnp.dot(f, w0_ref[...].astype(bf),
                             preferred_element_type=jnp.float32) + b0_ref[...])
        h = jnp.tanh(jnp.dot(h.astype(bf), w1_ref[...].astype(bf),
                             preferred_element_type=jnp.float32) + b1_ref[...])
        h = jnp.tanh(jnp.dot(h.astype(bf), w2_ref[...].astype(bf),
                             preferred_element_type=jnp.float32) + b2_ref[...])
        pred = jnp.dot(h.astype(bf), wi_ref[...].astype(bf),
                       preferred_element_type=jnp.float32) + bi_ref[...]
        pred_ref[...] = pred


def kernel(drug_graph, protein_graph, drug_embedding, protein_embedding,
           gene_embedding, W_gate, b_gate, W_exp, b_exp,
           W_out0, b_out0, W_out1, b_out1, W_out2, b_out2, W_int, b_int):
    x = jnp.concatenate([drug_graph, protein_graph, drug_embedding,
                         protein_embedding, gene_embedding], axis=1)

    grid = (E, NC)
    last = NC - 1

    def x_map(e, c):
        return 0, jnp.where(e == 0, c, last)

    def wg_map(e, c):
        return jnp.where(e == 0, c, last), 0

    pinned2 = lambda e, c: (0, 0)

    probs, pred = pl.pallas_call(
        _moe_kernel,
        grid=grid,
        in_specs=[
            pl.BlockSpec((B, DC), x_map),
            pl.BlockSpec((DC, E), wg_map),
            pl.BlockSpec((1, E), pinned2),
            pl.BlockSpec((1, DC, H), lambda e, c: (e, c, 0)),
            pl.BlockSpec((E, H), pinned2),
            pl.BlockSpec((512, 1024), pinned2),
            pl.BlockSpec((1, 1024), pinned2),
            pl.BlockSpec((1024, 512), pinned2),
            pl.BlockSpec((1, 512), pinned2),
            pl.BlockSpec((512, 256), pinned2),
            pl.BlockSpec((1, 256), pinned2),
            pl.BlockSpec((256, 2), pinned2),
            pl.BlockSpec((1, 2), pinned2),
        ],
        out_specs=[
            pl.BlockSpec((B, E), pinned2),
            pl.BlockSpec((B, 2), pinned2),
        ],
        out_shape=[
            jax.ShapeDtypeStruct((B, E), jnp.float32),
            jax.ShapeDtypeStruct((B, 2), jnp.float32),
        ],
        scratch_shapes=[
            pltpu.VMEM((B, D), jnp.bfloat16),
            pltpu.VMEM((B, E), jnp.float32),
            pltpu.VMEM((B, H), jnp.float32),
            pltpu.VMEM((B, H), jnp.float32),
            pltpu.VMEM((B, E), jnp.float32),
        ],
    )(x, W_gate, b_gate.reshape(1, E), W_exp, b_exp,
      W_out0, b_out0.reshape(1, 1024), W_out1, b_out1.reshape(1, 512),
      W_out2, b_out2.reshape(1, 256), W_int, b_int.reshape(1, 2))
    return (probs, pred)


# speedup vs baseline: 1.1957x; 1.0722x over previous
import functools

import jax
import jax.numpy as jnp
from jax.experimental import pallas as pl
from jax.experimental.pallas import tpu as pltpu

B = 1024
D = 8960
E = 6
H = 512
DC = 1280
NC = D // DC


def _moe_kernel(x_ref, wg_ref, bg_ref, wexp_ref, bexp_ref,
                w0_ref, b0_ref, w1_ref, b1_ref, w2_ref, b2_ref,
                wi_ref, bi_ref,
                probs_ref, pred_ref,
                xb_ref, logits_ref, acc_ref, fused_ref, wsel_ref):
    e = pl.program_id(0)
    c = pl.program_id(1)

    @pl.when(e == 0)
    def _gate_pass():
        xb_blk = x_ref[...]
        xb_ref[:, pl.ds(c * DC, DC)] = xb_blk
        part = jnp.dot(xb_blk, wg_ref[...].astype(jnp.bfloat16),
                       preferred_element_type=jnp.float32)
        prev = jnp.where(c == 0, jnp.zeros_like(logits_ref), logits_ref[...])
        logits_ref[...] = prev + part

    xb_c = xb_ref[:, pl.ds(c * DC, DC)]
    w_blk = wexp_ref[0].astype(jnp.bfloat16)
    prod = jnp.dot(xb_c, w_blk, preferred_element_type=jnp.float32)
    prev = jnp.where(c == 0, jnp.zeros_like(acc_ref), acc_ref[...])
    acc_ref[...] = prev + prod

    @pl.when(jnp.logical_and(e == 0, c == NC - 1))
    def _gate_epilogue():
        logits = logits_ref[...] + bg_ref[...]
        m = jnp.max(logits, axis=-1, keepdims=True)
        ex = jnp.exp(logits - m)
        probs = ex / jnp.sum(ex, axis=-1, keepdims=True)
        probs_ref[...] = probs
        iota = jax.lax.broadcasted_iota(jnp.int32, (B, E), 1)
        v1 = jnp.max(probs, axis=-1, keepdims=True)
        i1 = jnp.min(jnp.where(probs == v1, iota, E), axis=-1, keepdims=True)
        masked = jnp.where(iota == i1, -jnp.inf, probs)
        v2 = jnp.max(masked, axis=-1, keepdims=True)
        i2 = jnp.min(jnp.where(masked == v2, iota, E), axis=-1, keepdims=True)
        denom = v1 + v2 + 1e-9
        wsel_ref[...] = (jnp.where(iota == i1, v1 / denom, 0.0)
                         + jnp.where(iota == i2, v2 / denom, 0.0))

    @pl.when(c == NC - 1)
    def _expert_epilogue():
        row_iota = jax.lax.broadcasted_iota(jnp.int32, (E, H), 0)
        b_row = jnp.sum(jnp.where(row_iota == e, bexp_ref[...], 0.0),
                        axis=0, keepdims=True)
        eo = jnp.maximum(acc_ref[...] + b_row, 0.0)
        col_iota = jax.lax.broadcasted_iota(jnp.int32, (B, E), 1)
        w_col = jnp.sum(jnp.where(col_iota == e, wsel_ref[...], 0.0),
                        axis=-1, keepdims=True)
        prev = jnp.where(e == 0, jnp.zeros_like(fused_ref), fused_ref[...])
        fused_ref[...] = prev + eo * w_col

    @pl.when(jnp.logical_and(e == E - 1, c == NC - 1))
    def _mlp():
        bf = jnp.bfloat16
        f = fused_ref[...].astype(bf)
        h = jnp.tanh(jnp.dot(f, w0_ref[...].astype(bf),
                             preferred_element_type=jnp.float32) + b0_ref[...])
        h = jnp.tanh(jnp.dot(h.astype(bf), w1_ref[...].astype(bf),
                             preferred_element_type=jnp.float32) + b1_ref[...])
        h = jnp.tanh(jnp.dot(h.astype(bf), w2_ref[...].astype(bf),
                             preferred_element_type=jnp.float32) + b2_ref[...])
        pred = jnp.dot(h.astype(bf), wi_ref[...].astype(bf),
                       preferred_element_type=jnp.float32) + bi_ref[...]
        pred_ref[...] = pred


def kernel(drug_graph, protein_graph, drug_embedding, protein_embedding,
           gene_embedding, W_gate, b_gate, W_exp, b_exp,
           W_out0, b_out0, W_out1, b_out1, W_out2, b_out2, W_int, b_int):
    bf = jnp.bfloat16
    x = jnp.concatenate([drug_graph.astype(bf), protein_graph.astype(bf),
                         drug_embedding.astype(bf), protein_embedding.astype(bf),
                         gene_embedding.astype(bf)], axis=1)

    grid = (E, NC)
    last = NC - 1

    def x_map(e, c):
        return 0, jnp.where(e == 0, c, last)

    def wg_map(e, c):
        return jnp.where(e == 0, c, last), 0

    pinned2 = lambda e, c: (0, 0)

    probs, pred = pl.pallas_call(
        _moe_kernel,
        grid=grid,
        in_specs=[
            pl.BlockSpec((B, DC), x_map),
            pl.BlockSpec((DC, E), wg_map),
            pl.BlockSpec((1, E), pinned2),
            pl.BlockSpec((1, DC, H), lambda e, c: (e, c, 0)),
            pl.BlockSpec((E, H), pinned2),
            pl.BlockSpec((512, 1024), pinned2),
            pl.BlockSpec((1, 1024), pinned2),
            pl.BlockSpec((1024, 512), pinned2),
            pl.BlockSpec((1, 512), pinned2),
            pl.BlockSpec((512, 256), pinned2),
            pl.BlockSpec((1, 256), pinned2),
            pl.BlockSpec((256, 2), pinned2),
            pl.BlockSpec((1, 2), pinned2),
        ],
        out_specs=[
            pl.BlockSpec((B, E), pinned2),
            pl.BlockSpec((B, 2), pinned2),
        ],
        out_shape=[
            jax.ShapeDtypeStruct((B, E), jnp.float32),
            jax.ShapeDtypeStruct((B, 2), jnp.float32),
        ],
        scratch_shapes=[
            pltpu.VMEM((B, D), jnp.bfloat16),
            pltpu.VMEM((B, E), jnp.float32),
            pltpu.VMEM((B, H), jnp.float32),
            pltpu.VMEM((B, H), jnp.float32),
            pltpu.VMEM((B, E), jnp.float32),
        ],
    )(x, W_gate, b_gate.reshape(1, E), W_exp, b_exp,
      W_out0, b_out0.reshape(1, 1024), W_out1, b_out1.reshape(1, 512),
      W_out2, b_out2.reshape(1, 256), W_int, b_int.reshape(1, 2))
    return (probs, pred)


# speedup vs baseline: 1.2654x; 1.0583x over previous
import jax
import jax.numpy as jnp
from jax.experimental import pallas as pl
from jax.experimental.pallas import tpu as pltpu

B = 1024
D = 8960
E = 6
H = 512
HB = 256
NH = H // HB


def _moe_kernel(x_hbm, wg_ref, bg_ref, wexp_ref, bexp_ref,
                w0_ref, b0_ref, w1_ref, b1_ref, w2_ref, b2_ref,
                wi_ref, bi_ref,
                probs_ref, pred_ref,
                xb_ref, fused_ref, wsel_ref, sem):
    e = pl.program_id(0)
    h = pl.program_id(1)
    bf = jnp.bfloat16

    @pl.when(jnp.logical_and(e == 0, h == 0))
    def _load_and_gate():
        pltpu.make_async_copy(x_hbm, xb_ref, sem).start()
        pltpu.make_async_copy(x_hbm, xb_ref, sem).wait()
        logits = jnp.dot(xb_ref[...], wg_ref[...].astype(bf),
                         preferred_element_type=jnp.float32) + bg_ref[...]
        m = jnp.max(logits, axis=-1, keepdims=True)
        ex = jnp.exp(logits - m)
        probs = ex / jnp.sum(ex, axis=-1, keepdims=True)
        probs_ref[...] = probs
        iota = jax.lax.broadcasted_iota(jnp.int32, (B, E), 1)
        v1 = jnp.max(probs, axis=-1, keepdims=True)
        i1 = jnp.min(jnp.where(probs == v1, iota, E), axis=-1, keepdims=True)
        masked = jnp.where(iota == i1, -jnp.inf, probs)
        v2 = jnp.max(masked, axis=-1, keepdims=True)
        i2 = jnp.min(jnp.where(masked == v2, iota, E), axis=-1, keepdims=True)
        denom = v1 + v2 + 1e-9
        wsel_ref[...] = (jnp.where(iota == i1, v1 / denom, 0.0)
                         + jnp.where(iota == i2, v2 / denom, 0.0))

    out = jnp.dot(xb_ref[...], wexp_ref[0].astype(bf),
                  preferred_element_type=jnp.float32)
    row_iota = jax.lax.broadcasted_iota(jnp.int32, (E, HB), 0)
    b_row = jnp.sum(jnp.where(row_iota == e, bexp_ref[...], 0.0),
                    axis=0, keepdims=True)
    eo = jnp.maximum(out + b_row, 0.0)
    col_iota = jax.lax.broadcasted_iota(jnp.int32, (B, E), 1)
    w_col = jnp.sum(jnp.where(col_iota == e, wsel_ref[...], 0.0),
                    axis=-1, keepdims=True)
    hs = pl.multiple_of(h * HB, HB)
    prev = jnp.where(e == 0, jnp.zeros((B, HB), jnp.float32),
                     fused_ref[:, pl.ds(hs, HB)])
    fused_ref[:, pl.ds(hs, HB)] = prev + eo * w_col

    @pl.when(jnp.logical_and(e == E - 1, h == NH - 1))
    def _mlp():
        f = fused_ref[...].astype(bf)
        hid = jnp.tanh(jnp.dot(f, w0_ref[...].astype(bf),
                               preferred_element_type=jnp.float32) + b0_ref[...])
        hid = jnp.tanh(jnp.dot(hid.astype(bf), w1_ref[...].astype(bf),
                               preferred_element_type=jnp.float32) + b1_ref[...])
        hid = jnp.tanh(jnp.dot(hid.astype(bf), w2_ref[...].astype(bf),
                               preferred_element_type=jnp.float32) + b2_ref[...])
        pred_ref[...] = jnp.dot(hid.astype(bf), wi_ref[...].astype(bf),
                                preferred_element_type=jnp.float32) + bi_ref[...]


def kernel(drug_graph, protein_graph, drug_embedding, protein_embedding,
           gene_embedding, W_gate, b_gate, W_exp, b_exp,
           W_out0, b_out0, W_out1, b_out1, W_out2, b_out2, W_int, b_int):
    bf = jnp.bfloat16
    x = jnp.concatenate([drug_graph.astype(bf), protein_graph.astype(bf),
                         drug_embedding.astype(bf), protein_embedding.astype(bf),
                         gene_embedding.astype(bf)], axis=1)

    pinned2 = lambda e, h: (0, 0)

    probs, pred = pl.pallas_call(
        _moe_kernel,
        grid=(E, NH),
        in_specs=[
            pl.BlockSpec(memory_space=pl.ANY),
            pl.BlockSpec((D, E), pinned2),
            pl.BlockSpec((1, E), pinned2),
            pl.BlockSpec((1, D, HB), lambda e, h: (e, 0, h)),
            pl.BlockSpec((E, HB), lambda e, h: (0, h)),
            pl.BlockSpec((512, 1024), pinned2),
            pl.BlockSpec((1, 1024), pinned2),
            pl.BlockSpec((1024, 512), pinned2),
            pl.BlockSpec((1, 512), pinned2),
            pl.BlockSpec((512, 256), pinned2),
            pl.BlockSpec((1, 256), pinned2),
            pl.BlockSpec((256, 2), pinned2),
            pl.BlockSpec((1, 2), pinned2),
        ],
        out_specs=[
            pl.BlockSpec((B, E), pinned2),
            pl.BlockSpec((B, 2), pinned2),
        ],
        out_shape=[
            jax.ShapeDtypeStruct((B, E), jnp.float32),
            jax.ShapeDtypeStruct((B, 2), jnp.float32),
        ],
        scratch_shapes=[
            pltpu.VMEM((B, D), jnp.bfloat16),
            pltpu.VMEM((B, H), jnp.float32),
            pltpu.VMEM((B, E), jnp.float32),
            pltpu.SemaphoreType.DMA,
        ],
    )(x, W_gate, b_gate.reshape(1, E), W_exp, b_exp,
      W_out0, b_out0.reshape(1, 1024), W_out1, b_out1.reshape(1, 512),
      W_out2, b_out2.reshape(1, 256), W_int, b_int.reshape(1, 2))
    return (probs, pred)


# speedup vs baseline: 1.4700x; 1.1618x over previous
import jax
import jax.numpy as jnp
from jax.experimental import pallas as pl
from jax.experimental.pallas import tpu as pltpu

B = 1024
D = 8960
E = 6
H = 512
KC = 1792
NK = D // KC
FC = 896
NF = D // FC

_SEGS = ((0, 1024), (1024, 1280), (2304, 3072), (5376, 3072), (8448, 512))

_FILL = []
for _j in range(NF):
    _lo, _hi = _j * FC, (_j + 1) * FC
    _parts = []
    for _i, (_s, _w) in enumerate(_SEGS):
        a, b = max(_lo, _s), min(_hi, _s + _w)
        if a < b:
            _parts.append((_i, a - _s, b - a, a - _lo))
    _FILL.append(tuple(_parts))

_STEP_JS = tuple((2 * _k, 2 * _k + 1) for _k in range(NK))


def _moe_kernel(x0, x1, x2, x3, x4, wg_ref, bg_ref, wexp_ref, bexp_ref,
                w0_ref, b0_ref, w1_ref, b1_ref, w2_ref, b2_ref,
                wi_ref, bi_ref,
                probs_ref, pred_ref,
                xb_ref, acc_ref, fused_ref, wsel_ref,
                stg0, stg1, sem0, sem1):
    e = pl.program_id(0)
    k = pl.program_id(1)
    bf = jnp.bfloat16
    xrefs = (x0, x1, x2, x3, x4)
    stgs = (stg0, stg1)
    sems = (sem0, sem1)

    def copies(j):
        stg, sem = stgs[j % 2], sems[j % 2]
        return [pltpu.make_async_copy(
                    xrefs[i].at[:, pl.ds(src, w)],
                    stg.at[:, pl.ds(dst, w)], sem)
                for (i, src, w, dst) in _FILL[j]]

    def issue(j):
        for c in copies(j):
            c.start()

    def drain_and_cast(j):
        for c in copies(j):
            c.wait()
        xb_ref[:, pl.ds(j * FC, FC)] = stgs[j % 2][...].astype(bf)

    @pl.when(e == 0)
    def _fill():
        for kk, js in enumerate(_STEP_JS):
            @pl.when(k == kk)
            def _(js=js, kk=kk):
                if kk == 0:
                    issue(js[0])
                    issue(js[1])
                for j in js:
                    drain_and_cast(j)
                    if j + 2 < NF:
                        issue(j + 2)

    ks = pl.multiple_of(k * KC, KC)
    xb_c = xb_ref[:, pl.ds(ks, KC)]
    prod = jnp.dot(xb_c, wexp_ref[0].astype(bf),
                   preferred_element_type=jnp.float32)
    prev = jnp.where(k == 0, jnp.zeros_like(acc_ref), acc_ref[...])
    acc_ref[...] = prev + prod

    @pl.when(jnp.logical_and(e == 0, k == NK - 1))
    def _gate():
        logits = jnp.dot(xb_ref[...], wg_ref[...].astype(bf),
                         preferred_element_type=jnp.float32) + bg_ref[...]
        m = jnp.max(logits, axis=-1, keepdims=True)
        ex = jnp.exp(logits - m)
        probs = ex / jnp.sum(ex, axis=-1, keepdims=True)
        probs_ref[...] = probs
        iota = jax.lax.broadcasted_iota(jnp.int32, (B, E), 1)
        v1 = jnp.max(probs, axis=-1, keepdims=True)
        i1 = jnp.min(jnp.where(probs == v1, iota, E), axis=-1, keepdims=True)
        masked = jnp.where(iota == i1, -jnp.inf, probs)
        v2 = jnp.max(masked, axis=-1, keepdims=True)
        i2 = jnp.min(jnp.where(masked == v2, iota, E), axis=-1, keepdims=True)
        denom = v1 + v2 + 1e-9
        wsel_ref[...] = (jnp.where(iota == i1, v1 / denom, 0.0)
                         + jnp.where(iota == i2, v2 / denom, 0.0))

    @pl.when(k == NK - 1)
    def _expert_epilogue():
        row_iota = jax.lax.broadcasted_iota(jnp.int32, (E, H), 0)
        b_row = jnp.sum(jnp.where(row_iota == e, bexp_ref[...], 0.0),
                        axis=0, keepdims=True)
        eo = jnp.maximum(acc_ref[...] + b_row, 0.0)
        col_iota = jax.lax.broadcasted_iota(jnp.int32, (B, E), 1)
        w_col = jnp.sum(jnp.where(col_iota == e, wsel_ref[...], 0.0),
                        axis=-1, keepdims=True)
        prev = jnp.where(e == 0, jnp.zeros_like(fused_ref), fused_ref[...])
        fused_ref[...] = prev + eo * w_col

    @pl.when(jnp.logical_and(e == E - 1, k == NK - 1))
    def _mlp():
        f = fused_ref[...].astype(bf)
        hid = jnp.tanh(jnp.dot(f, w0_ref[...].astype(bf),
                               preferred_element_type=jnp.float32) + b0_ref[...])
        hid = jnp.tanh(jnp.dot(hid.astype(bf), w1_ref[...].astype(bf),
                               preferred_element_type=jnp.float32) + b1_ref[...])
        hid = jnp.tanh(jnp.dot(hid.astype(bf), w2_ref[...].astype(bf),
                               preferred_element_type=jnp.float32) + b2_ref[...])
        pred_ref[...] = jnp.dot(hid.astype(bf), wi_ref[...].astype(bf),
                                preferred_element_type=jnp.float32) + bi_ref[...]


def kernel(drug_graph, protein_graph, drug_embedding, protein_embedding,
           gene_embedding, W_gate, b_gate, W_exp, b_exp,
           W_out0, b_out0, W_out1, b_out1, W_out2, b_out2, W_int, b_int):
    pinned2 = lambda e, k: (0, 0)
    hbm = pl.BlockSpec(memory_space=pl.ANY)

    probs, pred = pl.pallas_call(
        _moe_kernel,
        grid=(E, NK),
        in_specs=[
            hbm, hbm, hbm, hbm, hbm,
            pl.BlockSpec((D, E), pinned2),
            pl.BlockSpec((1, E), pinned2),
            pl.BlockSpec((1, KC, H), lambda e, k: (e, k, 0)),
            pl.BlockSpec((E, H), pinned2),
            pl.BlockSpec((512, 1024), pinned2),
            pl.BlockSpec((1, 1024), pinned2),
            pl.BlockSpec((1024, 512), pinned2),
            pl.BlockSpec((1, 512), pinned2),
            pl.BlockSpec((512, 256), pinned2),
            pl.BlockSpec((1, 256), pinned2),
            pl.BlockSpec((256, 2), pinned2),
            pl.BlockSpec((1, 2), pinned2),
        ],
        out_specs=[
            pl.BlockSpec((B, E), pinned2),
            pl.BlockSpec((B, 2), pinned2),
        ],
        out_shape=[
            jax.ShapeDtypeStruct((B, E), jnp.float32),
            jax.ShapeDtypeStruct((B, 2), jnp.float32),
        ],
        scratch_shapes=[
            pltpu.VMEM((B, D), jnp.bfloat16),
            pltpu.VMEM((B, H), jnp.float32),
            pltpu.VMEM((B, H), jnp.float32),
            pltpu.VMEM((B, E), jnp.float32),
            pltpu.VMEM((B, FC), jnp.float32),
            pltpu.VMEM((B, FC), jnp.float32),
            pltpu.SemaphoreType.DMA,
            pltpu.SemaphoreType.DMA,
        ],
    )(drug_graph, protein_graph, drug_embedding, protein_embedding,
      gene_embedding, W_gate, b_gate.reshape(1, E), W_exp, b_exp,
      W_out0, b_out0.reshape(1, 1024), W_out1, b_out1.reshape(1, 512),
      W_out2, b_out2.reshape(1, 256), W_int, b_int.reshape(1, 2))
    return (probs, pred)
